# baseline (speedup 1.0000x reference)
import jax
import jax.numpy as jnp
from jax.experimental import pallas as pl
from jax.experimental.pallas import tpu as pltpu

_E = 64
_TOPK = 8
_BLK = 512


def _gate_one(hs, wt, idx_ref, w_ref, s):
    logits = jnp.dot(hs, wt, preferred_element_type=jnp.float32)
    rowmax = jnp.max(logits, axis=-1, keepdims=True)
    p = jnp.exp(logits - rowmax)
    b = p.shape[0]
    iota = jax.lax.broadcasted_iota(jnp.int32, (b, _E), 1)
    bits = jax.lax.bitcast_convert_type(p, jnp.int32)
    enc = ((bits & ~0x3F) | (_E - 1 - iota)) + 0x3F800000
    encf = jax.lax.bitcast_convert_type(enc, jnp.float32)
    vals = []
    keys = []
    for _ in range(_TOPK):
        mv = jnp.max(p, axis=-1, keepdims=True)
        cand = jnp.where(p == mv, encf, 0.0)
        m2 = jnp.max(cand, axis=-1, keepdims=True)
        keys.append(m2)
        vals.append(mv)
        kill = encf == m2
        p = jnp.where(kill, -1.0, p)
        encf = jnp.where(kill, 0.0, encf)
    v = jnp.concatenate(vals, axis=-1)
    kbits = jax.lax.bitcast_convert_type(
        jnp.concatenate(keys, axis=-1), jnp.int32)
    i = (_E - 1) - (kbits & 0x3F)
    denom = jnp.sum(v, axis=-1, keepdims=True) + 1e-20
    idx_ref[s] = i.T
    w_ref[s] = (v / denom).T


def _gate_kernel(hs0_ref, hs1_ref, wt_ref, idx_ref, w_ref):
    wt = wt_ref[...]
    _gate_one(hs0_ref[...], wt, idx_ref, w_ref, 0)
    _gate_one(hs1_ref[...], wt, idx_ref, w_ref, 1)


def kernel(hidden_states, weight):
    bsz, seq, h = hidden_states.shape
    t = bsz * seq
    hs = hidden_states.reshape(t, h)
    wt = weight.T
    half_blocks = t // (2 * _BLK)

    idx_t, w_t = pl.pallas_call(
        _gate_kernel,
        grid=(half_blocks,),
        in_specs=[
            pl.BlockSpec((_BLK, h), lambda i: (i, 0)),
            pl.BlockSpec((_BLK, h), lambda i: (i + half_blocks, 0)),
            pl.BlockSpec((h, _E), lambda i: (0, 0)),
        ],
        out_specs=[
            pl.BlockSpec((2, _TOPK, _BLK), lambda i: (0, 0, i)),
            pl.BlockSpec((2, _TOPK, _BLK), lambda i: (0, 0, i)),
        ],
        out_shape=[
            jax.ShapeDtypeStruct((2, _TOPK, t // 2), jnp.int32),
            jax.ShapeDtypeStruct((2, _TOPK, t // 2), jnp.float32),
        ],
        compiler_params=pltpu.CompilerParams(
            dimension_semantics=("parallel",)),
    )(hs, hs, wt)

    idx = jnp.concatenate([idx_t[0], idx_t[1]], axis=1).T
    w = jnp.concatenate([w_t[0], w_t[1]], axis=1).T
    return (idx, w)
